# baseline (speedup 1.0000x reference)
import functools

import jax
import jax.numpy as jnp
from jax import lax
from jax.experimental import pallas as pl
from jax.experimental.pallas import tpu as pltpu
from jax.experimental.pallas import tpu_sc as plsc

B = 16384
N_NUM = 128
N_CAT = 26
VOCAB = 100000

NW = 32
BPW = B // NW
LANES = 16
NVEC = BPW // LANES
CHUNK = 128
NCH = BPW // CHUNK


def _sc_embed_sum(xcat_t, tabs, off, ncat):
    mesh = plsc.VectorSubcoreMesh(core_axis_name="c", subcore_axis_name="s")

    @functools.partial(
        pl.kernel,
        mesh=mesh,
        out_type=jax.ShapeDtypeStruct((B,), jnp.float32),
        scratch_types=[
            pltpu.VMEM((ncat, BPW), jnp.int32),
            pltpu.VMEM((ncat, BPW), jnp.float32),
            pltpu.VMEM((BPW,), jnp.float32),
            pltpu.SemaphoreType.DMA,
        ],
    )
    def k(xcat_hbm, tab_hbm, out_hbm, idx_v, g_v, acc_v, sem):
        wid = lax.axis_index("s") * 2 + lax.axis_index("c")
        base = wid * BPW
        pltpu.sync_copy(
            xcat_hbm.at[pl.ds(off, ncat), pl.ds(base, BPW)], idx_v
        )

        for i in range(1, ncat):
            def obody(j, _, i=i):
                sl = pl.ds(j * LANES, LANES)
                idx_v[i, sl] = idx_v[i, sl] + (i * VOCAB)
                return 0
            lax.fori_loop(0, NVEC, obody, 0)

        copies = []
        for i in range(ncat):
            for c in range(NCH):
                sl = pl.ds(c * CHUNK, CHUNK)
                copies.append(
                    pltpu.async_copy(
                        tab_hbm.at[idx_v.at[i, sl]], g_v.at[i, sl], sem
                    )
                )
        for cp in copies:
            cp.wait()

        def rbody(j, _):
            sl = pl.ds(j * LANES, LANES)
            acc = g_v[0, sl]
            for i in range(1, ncat):
                acc = acc + g_v[i, sl]
            acc_v[sl] = acc
            return 0
        lax.fori_loop(0, NVEC, rbody, 0)

        pltpu.sync_copy(acc_v, out_hbm.at[pl.ds(base, BPW)])

    return k(xcat_t, tabs)


def _tc_linear_plus(x, w, b):
    blk = 2048

    def body(x_ref, w_ref, b_ref, o_ref):
        mv = jnp.dot(x_ref[...], w_ref[...], preferred_element_type=jnp.float32)
        o_ref[...] = mv.reshape(blk) + b_ref[0]

    return pl.pallas_call(
        body,
        grid=(B // blk,),
        in_specs=[
            pl.BlockSpec((blk, N_NUM), lambda i: (i, 0)),
            pl.BlockSpec((N_NUM, 1), lambda i: (0, 0)),
            pl.BlockSpec((1,), lambda i: (0,)),
        ],
        out_specs=pl.BlockSpec((blk,), lambda i: (i,)),
        out_shape=jax.ShapeDtypeStruct((B,), jnp.float32),
    )(x, w, b)


GROUPS = ((0, 8), (8, 8), (16, 10))


def kernel(x_num, x_cat, W, b, tables):
    xcat_t = x_cat.T
    embs = []
    for off, n in GROUPS:
        tcat = jnp.concatenate(
            [tables[i, :, 0] for i in range(off, off + n)]
        )
        embs.append(_sc_embed_sum(xcat_t, tcat, off, n))
    lin = _tc_linear_plus(x_num, W, b)
    out = lin + embs[0] + embs[1] + embs[2]
    return out[:, None]


# speedup vs baseline: 1.9988x; 1.9988x over previous
import functools

import jax
import jax.numpy as jnp
from jax import lax
from jax.experimental import pallas as pl
from jax.experimental.pallas import tpu as pltpu
from jax.experimental.pallas import tpu_sc as plsc

B = 16384
N_NUM = 128
N_CAT = 26
VOCAB = 100000

NW = 32
BPW = B // NW
LANES = 16
NVEC = BPW // LANES
CHUNK = 128
NCH = BPW // CHUNK


def _sc_embed_sum(xcat_t, tabs, off, ncat):
    mesh = plsc.VectorSubcoreMesh(core_axis_name="c", subcore_axis_name="s")

    @functools.partial(
        pl.kernel,
        mesh=mesh,
        out_type=jax.ShapeDtypeStruct((B,), jnp.float32),
        scratch_types=[
            pltpu.VMEM((ncat, BPW), jnp.int32),
            pltpu.VMEM((ncat, BPW), jnp.float32),
            pltpu.VMEM((BPW,), jnp.float32),
            pltpu.SemaphoreType.DMA,
        ],
    )
    def k(xcat_hbm, *rest):
        tab_hbms = rest[:ncat]
        out_hbm, idx_v, g_v, acc_v, sem = rest[ncat:]
        wid = lax.axis_index("s") * 2 + lax.axis_index("c")
        base = wid * BPW
        pltpu.sync_copy(
            xcat_hbm.at[pl.ds(off, ncat), pl.ds(base, BPW)], idx_v
        )

        copies = []
        for i in range(ncat):
            for c in range(NCH):
                sl = pl.ds(c * CHUNK, CHUNK)
                copies.append(
                    pltpu.async_copy(
                        tab_hbms[i].at[idx_v.at[i, sl]], g_v.at[i, sl], sem
                    )
                )
        for cp in copies:
            cp.wait()

        def rbody(j, _):
            sl = pl.ds(j * LANES, LANES)
            acc = g_v[0, sl]
            for i in range(1, ncat):
                acc = acc + g_v[i, sl]
            acc_v[sl] = acc
            return 0
        lax.fori_loop(0, NVEC, rbody, 0)

        pltpu.sync_copy(acc_v, out_hbm.at[pl.ds(base, BPW)])

    return k(xcat_t, *tabs)


def _tc_linear_plus(x, w, b):
    blk = 4096

    def body(x_ref, w_ref, b_ref, o_ref):
        mv = jnp.dot(x_ref[...], w_ref[...], preferred_element_type=jnp.float32)
        o_ref[...] = mv.reshape(blk) + b_ref[0]

    return pl.pallas_call(
        body,
        grid=(B // blk,),
        in_specs=[
            pl.BlockSpec((blk, N_NUM), lambda i: (i, 0)),
            pl.BlockSpec((N_NUM, 1), lambda i: (0, 0)),
            pl.BlockSpec((1,), lambda i: (0,)),
        ],
        out_specs=pl.BlockSpec((blk,), lambda i: (i,)),
        out_shape=jax.ShapeDtypeStruct((B,), jnp.float32),
    )(x, w, b)


GROUPS = ((0, 16), (16, 10))


def kernel(x_num, x_cat, W, b, tables):
    xcat_t = x_cat.T
    embs = []
    for off, n in GROUPS:
        g = [
            lax.slice(tables, (i, 0, 0), (i + 1, VOCAB, 1)).reshape(VOCAB)
            for i in range(off, off + n)
        ]
        embs.append(_sc_embed_sum(xcat_t, g, off, n))
    lin = _tc_linear_plus(x_num, W, b)
    out = lin + embs[0] + embs[1]
    return out[:, None]


# speedup vs baseline: 2.0183x; 1.0098x over previous
import functools

import jax
import jax.numpy as jnp
from jax import lax
from jax.experimental import pallas as pl
from jax.experimental.pallas import tpu as pltpu
from jax.experimental.pallas import tpu_sc as plsc

B = 16384
N_NUM = 128
N_CAT = 26
VOCAB = 100000

NW = 32
BPW = B // NW
LANES = 16
NVEC = BPW // LANES
CHUNK = 128
NCH = BPW // CHUNK


def _sc_embed_sum(xcat_t, tabs, off, ncat):
    mesh = plsc.VectorSubcoreMesh(core_axis_name="c", subcore_axis_name="s")

    @functools.partial(
        pl.kernel,
        mesh=mesh,
        out_type=jax.ShapeDtypeStruct((B,), jnp.float32),
        scratch_types=[
            pltpu.VMEM((ncat, BPW), jnp.int32),
            pltpu.VMEM((ncat, BPW), jnp.float32),
            pltpu.VMEM((BPW,), jnp.float32),
            pltpu.SemaphoreType.DMA,
        ],
    )
    def k(xcat_hbm, *rest):
        tab_hbms = rest[:ncat]
        out_hbm, idx_v, g_v, acc_v, sem = rest[ncat:]
        wid = lax.axis_index("s") * 2 + lax.axis_index("c")
        base = wid * BPW
        pltpu.sync_copy(
            xcat_hbm.at[pl.ds(off, ncat), pl.ds(base, BPW)], idx_v
        )

        copies = []
        for i in range(ncat):
            for c in range(NCH):
                sl = pl.ds(c * CHUNK, CHUNK)
                copies.append(
                    pltpu.async_copy(
                        tab_hbms[i].at[idx_v.at[i, sl]], g_v.at[i, sl], sem
                    )
                )
        for cp in copies:
            cp.wait()

        def rbody(j, _):
            sl = pl.ds(j * LANES, LANES)
            acc = g_v[0, sl]
            for i in range(1, ncat):
                acc = acc + g_v[i, sl]
            acc_v[sl] = acc
            return 0
        lax.fori_loop(0, NVEC, rbody, 0)

        pltpu.sync_copy(acc_v, out_hbm.at[pl.ds(base, BPW)])

    return k(xcat_t, *tabs)


def _tc_linear_plus(x, w, b):
    blk = 2048

    def body(x_ref, w_ref, b_ref, o_ref):
        mv = jnp.dot(x_ref[...], w_ref[...], preferred_element_type=jnp.float32)
        o_ref[...] = mv.reshape(blk) + b_ref[0]

    return pl.pallas_call(
        body,
        grid=(B // blk,),
        in_specs=[
            pl.BlockSpec((blk, N_NUM), lambda i: (i, 0)),
            pl.BlockSpec((N_NUM, 1), lambda i: (0, 0)),
            pl.BlockSpec((1,), lambda i: (0,)),
        ],
        out_specs=pl.BlockSpec((blk,), lambda i: (i,)),
        out_shape=jax.ShapeDtypeStruct((B,), jnp.float32),
    )(x, w, b)


GROUPS = ((0, 16), (16, 10))


def kernel(x_num, x_cat, W, b, tables):
    xcat_t = x_cat.T
    embs = []
    for off, n in GROUPS:
        g = [
            lax.slice(tables, (i, 0, 0), (i + 1, VOCAB, 1)).reshape(VOCAB)
            for i in range(off, off + n)
        ]
        embs.append(_sc_embed_sum(xcat_t, g, off, n))
    lin = _tc_linear_plus(x_num, W, b)
    out = lin + embs[0] + embs[1]
    return out[:, None]
